# baseline (speedup 1.0000x reference)
import functools

import jax
import jax.numpy as jnp
from jax import lax
from jax.experimental import pallas as pl
from jax.experimental.pallas import tpu as pltpu
from jax.experimental.pallas import tpu_sc as plsc

N = 32768
J, C = 24, 8
L = 128
JC = J * C
NG = 12
NC, NS = 2, 16
NW = NC * NS
PW = N // NW
CHUNK = 16
NCHUNK = PW // CHUNK
AUX = 2 * NG * CHUNK


def _coeff_body(qx_ref, qy_ref, qz_ref, s_ref, idx_ref, w_ref):
    comps = []
    for d, qr in enumerate((qx_ref, qy_ref, qz_ref)):
        qd = qr[...]
        qd = qd - jnp.mean(qd)
        comps.append(qd * (2.0 / s_ref[d]))
    x, y, z = comps
    planes = [(y, x), (z, y), (x, z)]
    for p, (gx, gy) in enumerate(planes):
        ix = (gx + 1.0) * (0.5 * (L - 1))
        iy = (gy + 1.0) * (0.5 * (L - 1))
        ix0 = jnp.floor(ix)
        iy0 = jnp.floor(iy)
        wx1 = ix - ix0
        wy1 = iy - iy0
        corners = [
            (iy0, ix0, (1.0 - wy1) * (1.0 - wx1)),
            (iy0, ix0 + 1.0, (1.0 - wy1) * wx1),
            (iy0 + 1.0, ix0, wy1 * (1.0 - wx1)),
            (iy0 + 1.0, ix0 + 1.0, wy1 * wx1),
        ]
        for k, (iy_, ix_, w) in enumerate(corners):
            valid = (ix_ >= 0) & (ix_ <= L - 1) & (iy_ >= 0) & (iy_ <= L - 1)
            cell = (jnp.clip(iy_, 0, L - 1).astype(jnp.int32) * L
                    + jnp.clip(ix_, 0, L - 1).astype(jnp.int32))
            idx_ref[p * 4 + k] = cell
            w_ref[p * 4 + k] = jnp.where(valid, w, 0.0)


def _coeffs(qx, qy, qz, scale):
    return pl.pallas_call(
        _coeff_body,
        out_shape=(
            jax.ShapeDtypeStruct((NG, N // 128, 128), jnp.int32),
            jax.ShapeDtypeStruct((NG, N // 128, 128), jnp.float32),
        ),
        in_specs=[
            pl.BlockSpec(memory_space=pltpu.VMEM),
            pl.BlockSpec(memory_space=pltpu.VMEM),
            pl.BlockSpec(memory_space=pltpu.VMEM),
            pl.BlockSpec(memory_space=pltpu.SMEM),
        ],
    )(qx, qy, qz, scale)


def _sc_sample(tx, ty, tz, auxf):
    mesh = plsc.VectorSubcoreMesh(core_axis_name="c", subcore_axis_name="s")

    @functools.partial(
        pl.kernel,
        out_type=jax.ShapeDtypeStruct((JC, N // 8, 3 * 8), jnp.float32),
        mesh=mesh,
        scratch_types=[
            [pltpu.VMEM((AUX,), jnp.int32) for _ in range(2)],
            [[pltpu.VMEM((4 * CHUNK, JC), jnp.float32) for _ in range(3)]
             for _ in range(2)],
            [pltpu.VMEM((JC, CHUNK // 8, 3 * 8), jnp.float32) for _ in range(2)],
            [pltpu.SemaphoreType.DMA for _ in range(2)],
            [pltpu.SemaphoreType.DMA for _ in range(2)],
            [pltpu.SemaphoreType.DMA for _ in range(2)],
        ],
        compiler_params=pltpu.CompilerParams(
            use_tc_tiling_on_sc=False, needs_layout_passes=False),
    )
    def k(tx_h, ty_h, tz_h, aux_h, out_h, aux_v, rows_v, tr_v, ssem, gsem,
          osem):
        wid = lax.axis_index("s") * NC + lax.axis_index("c")
        tables = (tx_h, ty_h, tz_h)
        lanes = lax.iota(jnp.int32, 16)
        dnums = lax.GatherDimensionNumbers(
            offset_dims=(), collapsed_slice_dims=(0,), start_index_map=(0,))

        def fire_stage(t, b):
            pltpu.async_copy(
                aux_h.at[pl.ds((wid * NCHUNK + t) * AUX, AUX)], aux_v[b],
                ssem[b])

        def drain_stage(b):
            pltpu.make_async_copy(
                aux_h.at[pl.ds(0, AUX)], aux_v[b], ssem[b]).wait()

        def fire_gathers(b):
            for p in range(3):
                pltpu.async_copy(
                    tables[p].at[aux_v[b].at[pl.ds(p * 4 * CHUNK, 4 * CHUNK)]],
                    rows_v[b][p], gsem[b])

        def drain_gathers(b):
            for p in range(3):
                pltpu.make_async_copy(
                    tables[p].at[pl.ds(0, 4 * CHUNK)], rows_v[b][p],
                    gsem[b]).wait()

        def drain_out(b):
            pltpu.make_async_copy(
                tr_v[b], out_h.at[:, pl.ds(0, CHUNK // 8)], osem[b]).wait()


        fire_stage(0, 0)
        fire_stage(1, 1)
        drain_stage(0)
        fire_gathers(0)

        def half(t, b):
            bn = b ^ 1
            @pl.when(t + 1 < NCHUNK)
            def _():
                drain_stage(bn)
                fire_gathers(bn)

            drain_gathers(b)

            @pl.when(t >= 2)
            def _():
                drain_out(b)

            wgs = [plsc.bitcast(aux_v[b][pl.ds(NG * CHUNK + g * CHUNK, 16)],
                                jnp.float32)
                   for g in range(NG)]

            def point(i, carry):
                ivec = jnp.full((16, 1), i, jnp.int32)
                g8 = jnp.full((16,), i // 8, jnp.int32)
                for p in range(3):
                    rp = rows_v[b][p]
                    cc = jnp.full((16,), p * 8 + i % 8, jnp.int32)
                    wv = [lax.gather(wgs[p * 4 + k], ivec, dnums,
                                     slice_sizes=(1,),
                                     mode=lax.GatherScatterMode.PROMISE_IN_BOUNDS)
                          for k in range(4)]
                    for v in range(JC // 16):
                        sl = pl.ds(v * 16, 16)
                        acc = (rp[i, sl] * wv[0]
                               + rp[CHUNK + i, sl] * wv[1]
                               + rp[2 * CHUNK + i, sl] * wv[2]
                               + rp[3 * CHUNK + i, sl] * wv[3])
                        plsc.store_scatter(
                            tr_v[b], [lanes + v * 16, g8, cc], acc)
                return carry

            lax.fori_loop(0, CHUNK, point, 0)
            g8b = (wid * PW + t * CHUNK) // 8
            pltpu.async_copy(tr_v[b], out_h.at[:, pl.ds(g8b, CHUNK // 8)],
                             osem[b])

            @pl.when(t + 2 < NCHUNK)
            def _():
                fire_stage(t + 2, b)

        def pair(t0, carry):
            half(2 * t0, 0)
            half(2 * t0 + 1, 1)
            return carry

        lax.fori_loop(0, NCHUNK // 2, pair, 0)
        drain_out(0)
        drain_out(1)

    return k(tx, ty, tz, auxf)


def kernel(id, query_points, scale, feat_lines_x, feat_lines_y, feat_lines_z):
    idq = jnp.asarray(id, jnp.int32)
    tabs = []
    for f in (feat_lines_x, feat_lines_y, feat_lines_z):
        t = lax.dynamic_index_in_dim(f, idq, axis=1, keepdims=False)
        tabs.append(jnp.transpose(t, (1, 2, 0, 3)).reshape(L * L, JC))
    qp = query_points.astype(jnp.float32)
    cols = [qp[:, d].reshape(N // 128, 128) for d in range(3)]
    idx12, w12 = _coeffs(cols[0], cols[1], cols[2], scale.astype(jnp.float32))
    aux = jnp.concatenate(
        [idx12.reshape(NG, N), lax.bitcast_convert_type(w12, jnp.int32).reshape(NG, N)],
        axis=0)
    auxf = aux.reshape(2 * NG, N // CHUNK, CHUNK).transpose(1, 0, 2).reshape(-1)
    g = _sc_sample(tabs[0], tabs[1], tabs[2], auxf)
    return g.reshape(1, N, 3 * JC)


# speedup vs baseline: 1.0250x; 1.0250x over previous
import functools

import jax
import jax.numpy as jnp
from jax import lax
from jax.experimental import pallas as pl
from jax.experimental.pallas import tpu as pltpu
from jax.experimental.pallas import tpu_sc as plsc

N = 32768
J, C = 24, 8
L = 128
JC = J * C
NG = 12
NC, NS = 2, 16
NW = NC * NS
PW = N // NW
CHUNK = 16
NCHUNK = PW // CHUNK
AUX = 2 * NG * CHUNK


def _coeff_body(qx_ref, qy_ref, qz_ref, s_ref, idx_ref, w_ref):
    comps = []
    for d, qr in enumerate((qx_ref, qy_ref, qz_ref)):
        qd = qr[...]
        qd = qd - jnp.mean(qd)
        comps.append(qd * (2.0 / s_ref[d]))
    x, y, z = comps
    planes = [(y, x), (z, y), (x, z)]
    for p, (gx, gy) in enumerate(planes):
        ix = (gx + 1.0) * (0.5 * (L - 1))
        iy = (gy + 1.0) * (0.5 * (L - 1))
        ix0 = jnp.floor(ix)
        iy0 = jnp.floor(iy)
        wx1 = ix - ix0
        wy1 = iy - iy0
        corners = [
            (iy0, ix0, (1.0 - wy1) * (1.0 - wx1)),
            (iy0, ix0 + 1.0, (1.0 - wy1) * wx1),
            (iy0 + 1.0, ix0, wy1 * (1.0 - wx1)),
            (iy0 + 1.0, ix0 + 1.0, wy1 * wx1),
        ]
        for k, (iy_, ix_, w) in enumerate(corners):
            valid = (ix_ >= 0) & (ix_ <= L - 1) & (iy_ >= 0) & (iy_ <= L - 1)
            cell = (jnp.clip(iy_, 0, L - 1).astype(jnp.int32) * L
                    + jnp.clip(ix_, 0, L - 1).astype(jnp.int32))
            idx_ref[p * 4 + k] = cell
            w_ref[p * 4 + k] = jnp.where(valid, w, 0.0)


def _coeffs(qx, qy, qz, scale):
    return pl.pallas_call(
        _coeff_body,
        out_shape=(
            jax.ShapeDtypeStruct((NG, N // 128, 128), jnp.int32),
            jax.ShapeDtypeStruct((NG, N // 128, 128), jnp.float32),
        ),
        in_specs=[
            pl.BlockSpec(memory_space=pltpu.VMEM),
            pl.BlockSpec(memory_space=pltpu.VMEM),
            pl.BlockSpec(memory_space=pltpu.VMEM),
            pl.BlockSpec(memory_space=pltpu.SMEM),
        ],
    )(qx, qy, qz, scale)


def _sc_sample(tx, ty, tz, auxf):
    mesh = plsc.VectorSubcoreMesh(core_axis_name="c", subcore_axis_name="s")

    @functools.partial(
        pl.kernel,
        out_type=jax.ShapeDtypeStruct((JC, N // 8, 3 * 8), jnp.float32),
        mesh=mesh,
        scratch_types=[
            [pltpu.VMEM((2 * NG, CHUNK), jnp.int32) for _ in range(2)],
            [[pltpu.VMEM((4 * CHUNK, JC), jnp.float32) for _ in range(3)]
             for _ in range(2)],
            [pltpu.VMEM((JC, CHUNK // 8, 3 * 8), jnp.float32) for _ in range(2)],
            [pltpu.SemaphoreType.DMA for _ in range(2)],
            [pltpu.SemaphoreType.DMA for _ in range(2)],
            [pltpu.SemaphoreType.DMA for _ in range(2)],
        ],
        compiler_params=pltpu.CompilerParams(
            use_tc_tiling_on_sc=False, needs_layout_passes=False),
    )
    def k(tx_h, ty_h, tz_h, aux_h, out_h, aux_v, rows_v, tr_v, ssem, gsem,
          osem):
        wid = lax.axis_index("s") * NC + lax.axis_index("c")
        tables = (tx_h, ty_h, tz_h)
        lanes = lax.iota(jnp.int32, 16)
        dnums = lax.GatherDimensionNumbers(
            offset_dims=(), collapsed_slice_dims=(0,), start_index_map=(0,))

        def fire_stage(t, b):
            pltpu.async_copy(
                aux_h.at[:, wid * NCHUNK + t, :], aux_v[b], ssem[b])

        def drain_stage(b):
            pltpu.make_async_copy(
                aux_h.at[:, 0, :], aux_v[b], ssem[b]).wait()

        def fire_gathers(b):
            for p in range(3):
                for g in range(4):
                    pltpu.async_copy(
                        tables[p].at[aux_v[b].at[p * 4 + g]],
                        rows_v[b][p].at[pl.ds(g * CHUNK, CHUNK)], gsem[b])

        def drain_gathers(b):
            for p in range(3):
                for g in range(4):
                    pltpu.make_async_copy(
                        tables[p].at[pl.ds(0, CHUNK)],
                        rows_v[b][p].at[pl.ds(g * CHUNK, CHUNK)],
                        gsem[b]).wait()

        def drain_out(b):
            pltpu.make_async_copy(
                tr_v[b], out_h.at[:, pl.ds(0, CHUNK // 8)], osem[b]).wait()


        fire_stage(0, 0)
        fire_stage(1, 1)
        drain_stage(0)
        fire_gathers(0)

        def half(t, b):
            bn = b ^ 1
            @pl.when(t + 1 < NCHUNK)
            def _():
                drain_stage(bn)
                fire_gathers(bn)

            drain_gathers(b)

            @pl.when(t >= 2)
            def _():
                drain_out(b)

            wgs = [plsc.bitcast(aux_v[b][NG + g, :], jnp.float32)
                   for g in range(NG)]

            def point(i, carry):
                ivec = jnp.full((16, 1), i, jnp.int32)
                g8 = jnp.full((16,), i // 8, jnp.int32)
                for p in range(3):
                    rp = rows_v[b][p]
                    cc = jnp.full((16,), p * 8 + i % 8, jnp.int32)
                    wv = [lax.gather(wgs[p * 4 + k], ivec, dnums,
                                     slice_sizes=(1,),
                                     mode=lax.GatherScatterMode.PROMISE_IN_BOUNDS)
                          for k in range(4)]
                    for v in range(JC // 16):
                        sl = pl.ds(v * 16, 16)
                        acc = (rp[i, sl] * wv[0]
                               + rp[CHUNK + i, sl] * wv[1]
                               + rp[2 * CHUNK + i, sl] * wv[2]
                               + rp[3 * CHUNK + i, sl] * wv[3])
                        plsc.store_scatter(
                            tr_v[b], [lanes + v * 16, g8, cc], acc)
                return carry

            lax.fori_loop(0, CHUNK, point, 0)
            g8b = (wid * PW + t * CHUNK) // 8
            pltpu.async_copy(tr_v[b], out_h.at[:, pl.ds(g8b, CHUNK // 8)],
                             osem[b])

            @pl.when(t + 2 < NCHUNK)
            def _():
                fire_stage(t + 2, b)

        def pair(t0, carry):
            half(2 * t0, 0)
            half(2 * t0 + 1, 1)
            return carry

        lax.fori_loop(0, NCHUNK // 2, pair, 0)
        drain_out(0)
        drain_out(1)

    return k(tx, ty, tz, auxf)


def kernel(id, query_points, scale, feat_lines_x, feat_lines_y, feat_lines_z):
    idq = jnp.asarray(id, jnp.int32)
    tabs = []
    for f in (feat_lines_x, feat_lines_y, feat_lines_z):
        t = lax.dynamic_index_in_dim(f, idq, axis=1, keepdims=False)
        tabs.append(jnp.transpose(t, (1, 2, 0, 3)).reshape(L * L, JC))
    qp = query_points.astype(jnp.float32)
    cols = [qp[:, d].reshape(N // 128, 128) for d in range(3)]
    idx12, w12 = _coeffs(cols[0], cols[1], cols[2], scale.astype(jnp.float32))
    auxf = jnp.concatenate(
        [idx12.reshape(NG, N), lax.bitcast_convert_type(w12, jnp.int32).reshape(NG, N)],
        axis=0).reshape(2 * NG, N // CHUNK, CHUNK)
    g = _sc_sample(tabs[0], tabs[1], tabs[2], auxf)
    return g.reshape(1, N, 3 * JC)
